# Initial kernel scaffold; baseline (speedup 1.0000x reference)
import jax
import jax.numpy as jnp
from jax.experimental import pallas as pl


def kernel(x, W1, b1, W2, b2, W3, b3):
    raise NotImplementedError("write your pallas kernel here")


# speedup vs baseline: 6.6534x; 6.6534x over previous
import functools

import jax
import jax.numpy as jnp
from jax.experimental import pallas as pl
from jax.experimental.pallas import tpu as pltpu

_N = 64
_K = 9
_CP = 128
_TILE = 256


def _layer_body(h_ref, xs_ref, xst_ref, wt_ref, b_ref, out_ref, y_ref, *,
                relu: bool):
    @pl.when(pl.program_id(1) == 0)
    def _():
        for k in range(_K):
            y_ref[k * _N:(k + 1) * _N, :] = jnp.dot(
                xs_ref[0].astype(jnp.bfloat16),
                wt_ref[k].astype(jnp.bfloat16),
                preferred_element_type=jnp.float32)

    xt = h_ref[0]
    xst = xst_ref[0]
    dot = jnp.dot(xt.astype(jnp.bfloat16), xst.astype(jnp.bfloat16),
                  preferred_element_type=jnp.float32)
    x2 = jnp.sum(xt * xt, axis=1, keepdims=True)
    s2 = jnp.sum(xst * xst, axis=0, keepdims=True)
    sim = -((x2 - 2.0 * dot) + s2)

    a = sim[:, None, :]
    bm = sim[:, :, None]
    tri = (jax.lax.broadcasted_iota(jnp.int32, (_N, _N), 0)
           < jax.lax.broadcasted_iota(jnp.int32, (_N, _N), 1))[None]
    cond = (bm > a) | ((bm == a) & tri)
    rank = jnp.sum(cond.astype(jnp.int32), axis=1)

    ranks_tiled = jnp.concatenate([rank] * _K, axis=1)
    kcol = jax.lax.broadcasted_iota(jnp.int32, (_TILE, _K * _N), 1) // _N
    p = (ranks_tiled == kcol).astype(jnp.float32)

    o = jnp.dot(p, y_ref[:, :], preferred_element_type=jnp.float32,
                precision=jax.lax.Precision.HIGHEST)
    o = o + b_ref[...]
    if relu:
        o = jnp.maximum(o, 0.0)
    out_ref[0] = o


def _layer(h, wl, bl, idx, relu):
    B, T, _ = h.shape
    co, c, _ = wl.shape
    xs = jnp.take(h, idx, axis=1)
    xst = xs.transpose(0, 2, 1)
    wt = jnp.pad(wl.transpose(2, 1, 0),
                 ((0, 0), (0, _CP - c), (0, _CP - co)))
    b_pad = jnp.pad(bl, (0, _CP - co)).reshape(1, _CP)
    grid = (B, T // _TILE)
    return pl.pallas_call(
        functools.partial(_layer_body, relu=relu),
        grid=grid,
        in_specs=[
            pl.BlockSpec((1, _TILE, _CP), lambda b, t: (b, t, 0)),
            pl.BlockSpec((1, _N, _CP), lambda b, t: (b, 0, 0)),
            pl.BlockSpec((1, _CP, _N), lambda b, t: (b, 0, 0)),
            pl.BlockSpec((_K, _CP, _CP), lambda b, t: (0, 0, 0)),
            pl.BlockSpec((1, _CP), lambda b, t: (0, 0)),
        ],
        out_specs=pl.BlockSpec((1, _TILE, _CP), lambda b, t: (b, t, 0)),
        out_shape=jax.ShapeDtypeStruct((B, T, _CP), jnp.float32),
        scratch_shapes=[pltpu.VMEM((_K * _N, _CP), jnp.float32)],
    )(h, xs, xst, wt, b_pad)


def kernel(x, W1, b1, W2, b2, W3, b3):
    B, C, H, W = x.shape
    r = 2
    xu = x.reshape(B, C, H // r, r, W // r, r).transpose(0, 1, 3, 5, 2, 4)
    xu = xu.reshape(B, C * r * r, (H // r) * (W // r))
    T = xu.shape[-1]
    h = jnp.pad(xu.transpose(0, 2, 1), ((0, 0), (0, 0), (0, _CP - C * r * r)))

    idx = jnp.linspace(0, T - 1, _N).astype(jnp.int32)
    h = _layer(h, W1, b1, idx, relu=True)
    h = _layer(h, W2, b2, idx, relu=True)
    h = _layer(h, W3, b3, idx, relu=False)

    co = W3.shape[0]
    out = h[:, :, :co].transpose(0, 2, 1).reshape(B, co, H // r, W // r)
    cf = co // (r * r)
    out = out.reshape(B, cf, r, r, H // r, W // r).transpose(0, 1, 4, 2, 5, 3)
    return out.reshape(B, cf, H, W)


# speedup vs baseline: 10.3671x; 1.5582x over previous
import functools

import jax
import jax.numpy as jnp
from jax.experimental import pallas as pl
from jax.experimental.pallas import tpu as pltpu
from jax.experimental.shard_map import shard_map
from jax.sharding import PartitionSpec as P

_N = 64
_K = 9
_CP = 128
_TILE = 256


def _layer_body(h_ref, xs_ref, xst_ref, wt_ref, b_ref, tri_ref, out_ref,
                y_ref, *, relu: bool):
    @pl.when(pl.program_id(1) == 0)
    def _():
        for k in range(_K):
            y_ref[k * _N:(k + 1) * _N, :] = jnp.dot(
                xs_ref[0].astype(jnp.bfloat16),
                wt_ref[k].astype(jnp.bfloat16),
                preferred_element_type=jnp.float32)

    xt = h_ref[0]
    xst = xst_ref[0]
    dot = jnp.dot(xt.astype(jnp.bfloat16), xst.astype(jnp.bfloat16),
                  preferred_element_type=jnp.float32)
    x2 = jnp.sum(xt * xt, axis=1, keepdims=True)
    s2 = jnp.sum(xst * xst, axis=0, keepdims=True)
    sim = -((x2 - 2.0 * dot) + s2)

    a = sim[:, None, :]
    bm = sim[:, :, None]
    del tri_ref
    tri = (jax.lax.broadcasted_iota(jnp.int32, (_N, _N), 0)
           < jax.lax.broadcasted_iota(jnp.int32, (_N, _N), 1))[None]
    cond = (bm > a) | ((bm == a) & tri)
    rank = jnp.sum(cond.astype(jnp.int32), axis=1)

    ranks_tiled = jnp.concatenate([rank] * _K, axis=1)
    kcol = jax.lax.broadcasted_iota(jnp.int32, (_TILE, _K * _N), 1) // _N
    p = (ranks_tiled == kcol).astype(jnp.float32)

    o = jnp.dot(p, y_ref[:, :], preferred_element_type=jnp.float32,
                precision=jax.lax.Precision.HIGHEST)
    o = o + b_ref[...]
    if relu:
        o = jnp.maximum(o, 0.0)
    out_ref[0] = o


def _layer(h, xs, wl, bl, relu):
    B, T, _ = h.shape
    co, c, _ = wl.shape
    xst = xs.transpose(0, 2, 1)
    wt = jnp.pad(wl.transpose(2, 1, 0),
                 ((0, 0), (0, _CP - c), (0, _CP - co)))
    b_pad = jnp.pad(bl, (0, _CP - co)).reshape(1, _CP)
    tri = (jnp.arange(_N)[:, None] < jnp.arange(_N)[None, :]).astype(jnp.int32)
    grid = (B, T // _TILE)
    return pl.pallas_call(
        functools.partial(_layer_body, relu=relu),
        grid=grid,
        in_specs=[
            pl.BlockSpec((1, _TILE, _CP), lambda b, t: (b, t, 0)),
            pl.BlockSpec((1, _N, _CP), lambda b, t: (b, 0, 0)),
            pl.BlockSpec((1, _CP, _N), lambda b, t: (b, 0, 0)),
            pl.BlockSpec((_K, _CP, _CP), lambda b, t: (0, 0, 0)),
            pl.BlockSpec((1, _CP), lambda b, t: (0, 0)),
            pl.BlockSpec((_N, _N), lambda b, t: (0, 0)),
        ],
        out_specs=pl.BlockSpec((1, _TILE, _CP), lambda b, t: (b, t, 0)),
        out_shape=jax.ShapeDtypeStruct((B, T, _CP), jnp.float32),
        scratch_shapes=[pltpu.VMEM((_K * _N, _CP), jnp.float32)],
    )(h, xs, xst, wt, b_pad, tri)


def _gather_anchors(h_local, idx, offset, t_local):
    local_pos = jnp.clip(idx - offset, 0, t_local - 1)
    xs_l = jnp.take(h_local, local_pos, axis=1)
    own = ((idx >= offset) & (idx < offset + t_local))[None, :, None]
    return jax.lax.psum(jnp.where(own, xs_l, 0.0), "tok")


def _network_sharded(h, W1, b1, W2, b2, W3, b3, idx):
    t_local = h.shape[1]
    offset = jax.lax.axis_index("tok") * t_local
    xs = _gather_anchors(h, idx, offset, t_local)
    h = _layer(h, xs, W1, b1, relu=True)
    xs = _gather_anchors(h, idx, offset, t_local)
    h = _layer(h, xs, W2, b2, relu=True)
    xs = _gather_anchors(h, idx, offset, t_local)
    h = _layer(h, xs, W3, b3, relu=False)
    return h


def kernel(x, W1, b1, W2, b2, W3, b3):
    B, C, H, W = x.shape
    r = 2
    xu = x.reshape(B, C, H // r, r, W // r, r).transpose(0, 1, 3, 5, 2, 4)
    xu = xu.reshape(B, C * r * r, (H // r) * (W // r))
    T = xu.shape[-1]
    h = jnp.pad(xu.transpose(0, 2, 1), ((0, 0), (0, 0), (0, _CP - C * r * r)))

    idx = jnp.linspace(0, T - 1, _N).astype(jnp.int32)

    devs = jax.devices()
    nd = len(devs) if T % (len(devs) * _TILE) == 0 else 1
    if nd > 1:
        mesh = jax.make_mesh((nd,), ("tok",), devices=devs[:nd])
        tok_s = jax.sharding.NamedSharding(mesh, P(None, "tok", None))
        rep_s = jax.sharding.NamedSharding(mesh, P())
        h = jax.reshard(h, tok_s)
        W1, b1, W2, b2, W3, b3 = (jax.reshard(v, rep_s)
                                  for v in (W1, b1, W2, b2, W3, b3))
        fn = shard_map(
            functools.partial(_network_sharded, idx=idx),
            mesh=mesh,
            in_specs=(P(None, "tok", None), P(), P(), P(), P(), P(), P()),
            out_specs=P(None, "tok", None),
            check_rep=False,
        )
        h = fn(h, W1, b1, W2, b2, W3, b3)
        h = jax.reshard(h[:, :, :W3.shape[0]], rep_s)
    else:
        xs = jnp.take(h, idx, axis=1)
        h = _layer(h, xs, W1, b1, relu=True)
        xs = jnp.take(h, idx, axis=1)
        h = _layer(h, xs, W2, b2, relu=True)
        xs = jnp.take(h, idx, axis=1)
        h = _layer(h, xs, W3, b3, relu=False)

    co = W3.shape[0]
    out = h[:, :, :co].transpose(0, 2, 1).reshape(B, co, H // r, W // r)
    cf = co // (r * r)
    out = out.reshape(B, cf, r, r, H // r, W // r).transpose(0, 1, 4, 2, 5, 3)
    return out.reshape(B, cf, H, W)
